# baseline (speedup 1.0000x reference)
import functools

import jax
import jax.numpy as jnp
from jax import lax
from jax.experimental import pallas as pl
from jax.experimental.pallas import tpu as pltpu
from jax.experimental.pallas import tpu_sc as plsc

_D = 768
_NPOS = 12
_NNEG = 12
_DEEP = 4
_PROJ = 896
_LSEQ = 77
_NSEG = 4
_ROWS = _NSEG * _LSEQ
_CHUNK = 16
_NWORK = -(-_ROWS // _CHUNK)
_PAD_ROWS = _NWORK * _CHUNK

_CTX_RUNS = (
    (0, 0 * _LSEQ + 1, 0, _NPOS),
    (0, 1 * _LSEQ + 1, 0, _NPOS),
    (1, 1 * _LSEQ + 1 + _NPOS, 0, _NNEG),
    (2, 2 * _LSEQ + 1, 0, _NPOS),
    (2, 3 * _LSEQ + 1, 0, _NPOS),
    (3, 3 * _LSEQ + 1 + _NPOS, 0, _NNEG),
)


def _worker_plan(w):
    lo, hi = w * _CHUNK, min((w + 1) * _CHUNK, _ROWS)
    ctx = []
    for src, g0, s0, n in _CTX_RUNS:
        a, b = max(lo, g0), min(hi, g0 + n)
        if a < b:
            ctx.append((src, a - lo, s0 + a - g0, b - a))
    outs = []
    for seg in range(_NSEG):
        a, b = max(lo, seg * _LSEQ), min(hi, (seg + 1) * _LSEQ)
        if a < b:
            outs.append((seg, a - lo, a - seg * _LSEQ, b - a))
    return ctx, outs


_PLANS = [_worker_plan(w) for w in range(_NWORK)]

_info = plsc.get_sparse_core_info()
_NC = _info.num_cores


@functools.partial(
    pl.kernel,
    mesh=plsc.VectorSubcoreMesh(core_axis_name="c", subcore_axis_name="s"),
    out_type=tuple(
        jax.ShapeDtypeStruct((_LSEQ, _D), jnp.float32) for _ in range(_NSEG)
    ),
    scratch_types=[
        pltpu.VMEM((_CHUNK,), jnp.int32),
        pltpu.VMEM((_CHUNK, _D), jnp.float32),
        pltpu.SemaphoreType.DMA,
    ],
    compiler_params=pltpu.CompilerParams(use_tc_tiling_on_sc=False),
)
def _sc_prompts(table, gpos, gneg, lpos, lneg, tok,
                out_gp, out_gn, out_lp, out_ln, idx_v, rows_v, sem):
    ctx_refs = (gpos, gneg, lpos, lneg)
    out_refs = (out_gp, out_gn, out_lp, out_ln)
    wid = lax.axis_index("s") * _NC + lax.axis_index("c")
    for w in range(_NWORK):
        @pl.when(wid == w)
        def _():
            pltpu.sync_copy(tok.at[pl.ds(w * _CHUNK, _CHUNK)], idx_v)
            pltpu.async_copy(table.at[idx_v], rows_v, sem).wait()
            ctx, outs = _PLANS[w]
            for src, l0, s0, n in ctx:
                pltpu.sync_copy(ctx_refs[src].at[pl.ds(s0, n)],
                                rows_v.at[pl.ds(l0, n)])
            for seg, l0, r0, n in outs:
                pltpu.sync_copy(rows_v.at[pl.ds(l0, n)],
                                out_refs[seg].at[pl.ds(r0, n)])


def _proj_body(cp_ref, w_ref, b_ref, out_ref):
    out_ref[...] = (
        jnp.dot(cp_ref[0], w_ref[0], preferred_element_type=jnp.float32)
        + b_ref[0]
    )[None]


_proj = pl.pallas_call(
    _proj_body,
    grid=(8,),
    in_specs=[
        pl.BlockSpec((1, _DEEP, _D), lambda l: (l, 0, 0)),
        pl.BlockSpec((1, _D, _PROJ), lambda l: (l, 0, 0)),
        pl.BlockSpec((1, 1, _PROJ), lambda l: (l, 0, 0)),
    ],
    out_specs=pl.BlockSpec((1, _DEEP, _PROJ), lambda l: (l, 0, 0)),
    out_shape=jax.ShapeDtypeStruct((8, _DEEP, _PROJ), jnp.float32),
)


def kernel(token_embedding, ctx_global_pos, ctx_global_neg, ctx_local_pos,
           ctx_local_neg, compound_prompts_text, proj_W, proj_b,
           tokens_global_pos, tokens_global_neg, tokens_local_pos,
           tokens_local_neg):
    tok = jnp.concatenate([
        tokens_global_pos.reshape(-1), tokens_global_neg.reshape(-1),
        tokens_local_pos.reshape(-1), tokens_local_neg.reshape(-1),
        jnp.zeros((_PAD_ROWS - _ROWS,), jnp.int32),
    ])
    gp, gn, lp, ln = _sc_prompts(
        token_embedding,
        ctx_global_pos.reshape(_NPOS, _D), ctx_global_neg.reshape(_NNEG, _D),
        ctx_local_pos.reshape(_NPOS, _D), ctx_local_neg.reshape(_NNEG, _D),
        tok,
    )
    projected = _proj(compound_prompts_text, proj_W,
                      proj_b.reshape(8, 1, _PROJ))
    return (gp.reshape(1, _LSEQ, _D), gn.reshape(1, _LSEQ, _D),
            lp.reshape(1, _LSEQ, _D), ln.reshape(1, _LSEQ, _D), projected)


# speedup vs baseline: 4.3416x; 4.3416x over previous
import functools

import jax
import jax.numpy as jnp
import numpy as np
from jax import lax
from jax.experimental import pallas as pl
from jax.experimental.pallas import tpu as pltpu
from jax.experimental.pallas import tpu_sc as plsc

_D = 768
_NPOS = 12
_NNEG = 12
_DEEP = 4
_PROJ = 896
_LSEQ = 77
_NSEG = 4
_SEG_STRIDE = 80
_CHUNK = 16
_N_TOK_W = 15
_N_CTX_W = 5
_NWORK = _N_TOK_W + _N_CTX_W
_OUT_ROWS = _NSEG * _SEG_STRIDE

_N_CTX_SEG = (_NPOS, _NPOS + _NNEG, _NPOS, _NPOS + _NNEG)
_CTX_CAT_BASE = (0, 0, _NPOS + _NNEG, _NPOS + _NNEG)


def _build_jobs():
    tok_pos, tok_dst = [], []
    ctx_src, ctx_dst = [], []
    junk = []
    for seg in range(_NSEG):
        nctx = _N_CTX_SEG[seg]
        for pos in range(_LSEQ):
            if pos == 0 or pos > nctx:
                tok_pos.append(_LSEQ * seg + pos)
                tok_dst.append(_SEG_STRIDE * seg + pos)
            else:
                ctx_src.append(_CTX_CAT_BASE[seg] + pos - 1)
                ctx_dst.append(_SEG_STRIDE * seg + pos)
        junk.extend(_SEG_STRIDE * seg + r for r in range(_LSEQ, _SEG_STRIDE))
    tok_pad = _N_TOK_W * _CHUNK - len(tok_pos)
    ctx_pad = _N_CTX_W * _CHUNK - len(ctx_src)
    assert tok_pad + ctx_pad == len(junk)
    dst = tok_dst + junk[:tok_pad] + ctx_dst + junk[tok_pad:]
    src_ctx_part = ctx_src + [0] * ctx_pad
    return (np.asarray(tok_pos, np.int32), np.asarray(src_ctx_part, np.int32),
            np.asarray(dst, np.int32), tok_pad)


_TOK_POS, _CTX_SRC_PAD, _DST, _TOK_PAD = _build_jobs()

_info = plsc.get_sparse_core_info()
_NC = _info.num_cores


@functools.partial(
    pl.kernel,
    mesh=plsc.VectorSubcoreMesh(core_axis_name="c", subcore_axis_name="s"),
    out_type=jax.ShapeDtypeStruct((_OUT_ROWS, _D), jnp.float32),
    scratch_types=[
        pltpu.VMEM((_CHUNK,), jnp.int32),
        pltpu.VMEM((_CHUNK,), jnp.int32),
        pltpu.VMEM((_CHUNK, _D), jnp.float32),
        pltpu.SemaphoreType.DMA,
    ],
)
def _sc_prompts(table, ctx_cat, src_hbm, dst_hbm, out,
                sidx_v, didx_v, buf, sem):
    wid = lax.axis_index("s") * _NC + lax.axis_index("c")

    @pl.when(wid < _NWORK)
    def _():
        base = pl.multiple_of(wid * _CHUNK, _CHUNK)
        pltpu.sync_copy(src_hbm.at[pl.ds(base, _CHUNK)], sidx_v)
        pltpu.sync_copy(dst_hbm.at[pl.ds(base, _CHUNK)], didx_v)

        @pl.when(wid < _N_TOK_W)
        def _():
            pltpu.async_copy(table.at[sidx_v], buf, sem).wait()

        @pl.when(wid >= _N_TOK_W)
        def _():
            pltpu.async_copy(ctx_cat.at[sidx_v], buf, sem).wait()

        pltpu.async_copy(buf, out.at[didx_v], sem).wait()


def _proj_body(cp_ref, w_ref, b_ref, out_ref):
    out_ref[...] = (
        jnp.dot(cp_ref[0], w_ref[0], preferred_element_type=jnp.float32)
        + b_ref[0]
    )[None]


_proj = pl.pallas_call(
    _proj_body,
    grid=(8,),
    in_specs=[
        pl.BlockSpec((1, _DEEP, _D), lambda l: (l, 0, 0)),
        pl.BlockSpec((1, _D, _PROJ), lambda l: (l, 0, 0)),
        pl.BlockSpec((1, 1, _PROJ), lambda l: (l, 0, 0)),
    ],
    out_specs=pl.BlockSpec((1, _DEEP, _PROJ), lambda l: (l, 0, 0)),
    out_shape=jax.ShapeDtypeStruct((8, _DEEP, _PROJ), jnp.float32),
)


def kernel(token_embedding, ctx_global_pos, ctx_global_neg, ctx_local_pos,
           ctx_local_neg, compound_prompts_text, proj_W, proj_b,
           tokens_global_pos, tokens_global_neg, tokens_local_pos,
           tokens_local_neg):
    tok_flat = jnp.concatenate([
        tokens_global_pos.reshape(-1), tokens_global_neg.reshape(-1),
        tokens_local_pos.reshape(-1), tokens_local_neg.reshape(-1),
    ])
    src_idx = jnp.concatenate([
        tok_flat[jnp.asarray(_TOK_POS)],
        jnp.zeros((_TOK_PAD,), jnp.int32),
        jnp.asarray(_CTX_SRC_PAD),
    ])
    ctx_cat = jnp.concatenate([
        ctx_global_pos.reshape(_NPOS, _D), ctx_global_neg.reshape(_NNEG, _D),
        ctx_local_pos.reshape(_NPOS, _D), ctx_local_neg.reshape(_NNEG, _D),
    ])
    out = _sc_prompts(token_embedding, ctx_cat, src_idx, jnp.asarray(_DST))
    projected = _proj(compound_prompts_text, proj_W,
                      proj_b.reshape(8, 1, _PROJ))
    prompts = tuple(
        lax.slice(out, (_SEG_STRIDE * s, 0),
                  (_SEG_STRIDE * s + _LSEQ, _D)).reshape(1, _LSEQ, _D)
        for s in range(_NSEG)
    )
    return (*prompts, projected)
